# Initial kernel scaffold; baseline (speedup 1.0000x reference)
import jax
import jax.numpy as jnp
from jax.experimental import pallas as pl


def kernel(edge_index_list, x, W1, b1, W2, b2, W3, b3):
    raise NotImplementedError("write your pallas kernel here")


# speedup vs baseline: 8.9905x; 8.9905x over previous
import functools

import jax
import jax.numpy as jnp
from jax import lax
from jax.experimental import pallas as pl
from jax.experimental.pallas import tpu as pltpu
from jax.experimental.pallas import tpu_sc as plsc

N = 10000
E = 320000
D = 128
DH = D // 2
NC = 2
NS = 16
NW = NC * NS
EPT = E // NS
CH = 80
NCHUNK = EPT // CH

ACC_STRIPE = 632
ACC_ROWS = NS * ACC_STRIPE
DEG_STRIPE = 1880
DEG_ROWS = NS * DEG_STRIPE

_mesh = plsc.VectorSubcoreMesh(
    core_axis_name="c", subcore_axis_name="s", num_cores=NC, num_subcores=NS
)
_sc_params = pltpu.CompilerParams(use_tc_tiling_on_sc=False)


@functools.partial(
    pl.kernel,
    out_type=jax.ShapeDtypeStruct((NC, DEG_ROWS, 8), jnp.float32),
    mesh=_mesh,
    compiler_params=_sc_params,
    scratch_types=[
        pltpu.VMEM((CH,), jnp.int32),
        pltpu.VMEM((CH, 8), jnp.float32),
        pltpu.VMEM((DEG_STRIPE, 8), jnp.float32),
        pltpu.VMEM_SHARED((DEG_ROWS, 8), jnp.float32),
    ],
)
def _deg_kernel(dst_hbm, ones_hbm, z_hbm, deg_out, dstv, onesv, stage, deg_s):
    c = lax.axis_index("c")
    s = lax.axis_index("s")
    wid = c * NS + s
    pltpu.sync_copy(z_hbm, stage)
    pltpu.sync_copy(stage, deg_s.at[pl.ds(s * DEG_STRIPE, DEG_STRIPE)])
    pltpu.sync_copy(ones_hbm, onesv)
    plsc.subcore_barrier()
    nch = 3 * E // (NW * CH)
    base = wid * nch * CH

    def body(i, carry):
        pltpu.sync_copy(dst_hbm.at[pl.ds(base + i * CH, CH)], dstv)
        pltpu.sync_copy(onesv, deg_s.at[dstv], add=True)
        return carry

    lax.fori_loop(0, nch, body, 0)
    plsc.subcore_barrier()
    pltpu.sync_copy(deg_s.at[pl.ds(s * DEG_STRIPE, DEG_STRIPE)], stage)
    pltpu.sync_copy(stage, deg_out.at[c, pl.ds(s * DEG_STRIPE, DEG_STRIPE)])


@functools.partial(
    pl.kernel,
    out_type=jax.ShapeDtypeStruct((NC, ACC_ROWS, DH), jnp.float32),
    mesh=_mesh,
    compiler_params=_sc_params,
    scratch_types=[
        pltpu.VMEM((CH,), jnp.int32),
        pltpu.VMEM((CH,), jnp.int32),
        pltpu.VMEM((CH, DH), jnp.float32),
        pltpu.VMEM((ACC_STRIPE, DH), jnp.float32),
        pltpu.VMEM_SHARED((ACC_ROWS, DH), jnp.float32),
        pltpu.SemaphoreType.DMA,
    ],
)
def _msg_kernel(hp_hbm, src_hbm, dst_hbm, z_hbm, acc_out, srcv, dstv, rows, stage, acc_s, gsem):
    c = lax.axis_index("c")
    s = lax.axis_index("s")
    pltpu.sync_copy(z_hbm, stage)
    pltpu.sync_copy(stage, acc_s.at[pl.ds(s * ACC_STRIPE, ACC_STRIPE)])
    plsc.subcore_barrier()
    base = s * EPT
    table = hp_hbm.at[c]

    def body(i, carry):
        off = base + i * CH
        pltpu.sync_copy(src_hbm.at[pl.ds(off, CH)], srcv)
        pltpu.sync_copy(dst_hbm.at[pl.ds(off, CH)], dstv)
        pltpu.async_copy(table.at[srcv], rows, gsem).wait()
        pltpu.sync_copy(rows, acc_s.at[dstv], add=True)
        return carry

    lax.fori_loop(0, NCHUNK, body, 0)
    plsc.subcore_barrier()
    pltpu.sync_copy(acc_s.at[pl.ds(s * ACC_STRIPE, ACC_STRIPE)], stage)
    pltpu.sync_copy(stage, acc_out.at[c, pl.ds(s * ACC_STRIPE, ACC_STRIPE)])


BM = 2000


def _dis_body(deg8_ref, dis_ref):
    d = jnp.sum(deg8_ref[...], axis=(0, 3)) + 1.0
    dis_ref[...] = lax.rsqrt(d)


def _dis_call(deg8):
    return pl.pallas_call(
        _dis_body,
        grid=(N // BM,),
        in_specs=[pl.BlockSpec((NC, BM, 3, 8), lambda j: (0, j, 0, 0))],
        out_specs=pl.BlockSpec((BM, 3), lambda j: (j, 0)),
        out_shape=jax.ShapeDtypeStruct((N, 3), jnp.float32),
    )(deg8)


def _split(h):
    return h[:, :DH], h[:, DH:]


def _first_body(x_ref, w_ref, dis_ref, out_ref):
    h = jnp.dot(x_ref[...], w_ref[...], preferred_element_type=jnp.float32)
    h = h * dis_ref[:, 0:1]
    lo, hi = _split(h)
    out_ref[0] = lo
    out_ref[1] = hi


def _first_call(x, W1, dis):
    return pl.pallas_call(
        _first_body,
        grid=(N // BM,),
        in_specs=[
            pl.BlockSpec((BM, D), lambda j: (j, 0)),
            pl.BlockSpec((D, D), lambda j: (0, 0)),
            pl.BlockSpec((BM, 3), lambda j: (j, 0)),
        ],
        out_specs=pl.BlockSpec((NC, BM, DH), lambda j: (0, j, 0)),
        out_shape=jax.ShapeDtypeStruct((NC, N, DH), jnp.float32),
    )(x, W1, dis)


def _mid_body(l, acc_ref, hp_ref, dis_ref, b_ref, w_ref, out_ref):
    t = jnp.concatenate(
        [acc_ref[0] + hp_ref[0], acc_ref[1] + hp_ref[1]], axis=1
    )
    t = t * dis_ref[:, l : l + 1] + b_ref[...]
    t = jnp.maximum(t, 0.0)
    h = jnp.dot(t, w_ref[...], preferred_element_type=jnp.float32)
    h = h * dis_ref[:, l + 1 : l + 2]
    lo, hi = _split(h)
    out_ref[0] = lo
    out_ref[1] = hi


def _mid_call(l, acc, hp, dis, b, Wn):
    return pl.pallas_call(
        functools.partial(_mid_body, l),
        grid=(N // BM,),
        in_specs=[
            pl.BlockSpec((NC, BM, DH), lambda j: (0, j, 0)),
            pl.BlockSpec((NC, BM, DH), lambda j: (0, j, 0)),
            pl.BlockSpec((BM, 3), lambda j: (j, 0)),
            pl.BlockSpec((1, D), lambda j: (0, 0)),
            pl.BlockSpec((D, D), lambda j: (0, 0)),
        ],
        out_specs=pl.BlockSpec((NC, BM, DH), lambda j: (0, j, 0)),
        out_shape=jax.ShapeDtypeStruct((NC, N, DH), jnp.float32),
    )(acc, hp, dis, b, Wn)


def _last_body(acc_ref, hp_ref, dis_ref, b_ref, out_ref):
    t = jnp.concatenate(
        [acc_ref[0] + hp_ref[0], acc_ref[1] + hp_ref[1]], axis=1
    )
    t = t * dis_ref[:, 2:3] + b_ref[...]
    n2 = jnp.sum(t * t, axis=1, keepdims=True)
    out_ref[...] = t * lax.rsqrt(jnp.maximum(n2, 1e-24))


def _last_call(acc, hp, dis, b):
    return pl.pallas_call(
        _last_body,
        grid=(N // BM,),
        in_specs=[
            pl.BlockSpec((NC, BM, DH), lambda j: (0, j, 0)),
            pl.BlockSpec((NC, BM, DH), lambda j: (0, j, 0)),
            pl.BlockSpec((BM, 3), lambda j: (j, 0)),
            pl.BlockSpec((1, D), lambda j: (0, 0)),
        ],
        out_specs=pl.BlockSpec((BM, D), lambda j: (j, 0)),
        out_shape=jax.ShapeDtypeStruct((N, D), jnp.float32),
    )(acc, hp, dis, b)


def kernel(edge_index_list, x, W1, b1, W2, b2, W3, b3):
    src = edge_index_list[:, 0, :]
    dst = edge_index_list[:, 1, :]
    dst_off = (dst * 3 + jnp.arange(3, dtype=jnp.int32)[:, None]).reshape(3 * E)
    eighth = jnp.full((CH, 8), 0.125, jnp.float32)
    z8 = jnp.zeros((DEG_STRIPE, 8), jnp.float32)
    zrows = jnp.zeros((ACC_STRIPE, DH), jnp.float32)

    deg8 = _deg_kernel(dst_off, eighth, z8)[:, : 3 * N].reshape(NC, N, 3, 8)
    dis = _dis_call(deg8)

    hp = _first_call(x, W1, dis)
    acc = _msg_kernel(hp, src[0], dst[0], zrows)
    hp = _mid_call(0, acc, hp, dis, b1.reshape(1, D), W2)
    acc = _msg_kernel(hp, src[1], dst[1], zrows)
    hp = _mid_call(1, acc, hp, dis, b2.reshape(1, D), W3)
    acc = _msg_kernel(hp, src[2], dst[2], zrows)
    return _last_call(acc, hp, dis, b3.reshape(1, D))


# speedup vs baseline: 23.1724x; 2.5774x over previous
import functools

import jax
import jax.numpy as jnp
from jax import lax
from jax.experimental import pallas as pl
from jax.experimental.pallas import tpu as pltpu
from jax.experimental.pallas import tpu_sc as plsc

N = 10000
E = 320000
D = 128
DH = D // 2
NC = 2
NS = 16
NW = NC * NS
EPT = E // NS
CH = 400
NCHUNK = EPT // CH
NPAIR = NCHUNK // 2

ACC_STRIPE = 632
ACC_ROWS = NS * ACC_STRIPE
DEG_STRIPE = 1880
DEG_ROWS = NS * DEG_STRIPE

_mesh = plsc.VectorSubcoreMesh(
    core_axis_name="c", subcore_axis_name="s", num_cores=NC, num_subcores=NS
)
_sc_params = pltpu.CompilerParams(use_tc_tiling_on_sc=False)


CHD = 1200
DEG_EPT = 3 * E // NW
NCHD = DEG_EPT // CHD


@functools.partial(
    pl.kernel,
    out_type=jax.ShapeDtypeStruct((NC, DEG_ROWS, 8), jnp.float32),
    mesh=_mesh,
    compiler_params=_sc_params,
    scratch_types=[
        pltpu.VMEM((DEG_EPT,), jnp.int32),
        pltpu.VMEM((CHD, 8), jnp.float32),
        pltpu.VMEM((DEG_STRIPE, 8), jnp.float32),
        pltpu.VMEM_SHARED((DEG_ROWS, 8), jnp.float32),
        pltpu.SemaphoreType.DMA,
    ],
)
def _deg_kernel(dst_hbm, ones_hbm, z_hbm, deg_out, dstall, onesv, stage, deg_s, ssem):
    c = lax.axis_index("c")
    s = lax.axis_index("s")
    wid = c * NS + s
    pltpu.sync_copy(z_hbm, stage)
    pltpu.sync_copy(stage, deg_s.at[pl.ds(s * DEG_STRIPE, DEG_STRIPE)])
    pltpu.sync_copy(ones_hbm, onesv)
    pltpu.sync_copy(dst_hbm.at[pl.ds(wid * DEG_EPT, DEG_EPT)], dstall)
    plsc.subcore_barrier()

    def fire(i, carry):
        off = pl.multiple_of(i * CHD, 8)
        pltpu.async_copy(onesv, deg_s.at[dstall.at[pl.ds(off, CHD)]], ssem, add=True)
        return carry

    lax.fori_loop(0, NCHD, fire, 0)

    def drain(i, carry):
        pltpu.make_async_copy(onesv, deg_s.at[dstall.at[pl.ds(0, CHD)]], ssem).wait()
        return carry

    lax.fori_loop(0, NCHD, drain, 0)
    plsc.subcore_barrier()
    pltpu.sync_copy(deg_s.at[pl.ds(s * DEG_STRIPE, DEG_STRIPE)], stage)
    pltpu.sync_copy(stage, deg_out.at[c, pl.ds(s * DEG_STRIPE, DEG_STRIPE)])


STG2 = ACC_STRIPE - CH


@functools.partial(
    pl.kernel,
    out_type=jax.ShapeDtypeStruct((NC, ACC_ROWS, DH), jnp.float32),
    mesh=_mesh,
    compiler_params=_sc_params,
    scratch_types=[
        pltpu.VMEM((EPT,), jnp.int32),
        pltpu.VMEM((2, CH), jnp.int32),
        pltpu.VMEM((2, CH, DH), jnp.float32),
        pltpu.VMEM_SHARED((ACC_ROWS, DH), jnp.float32),
        pltpu.SemaphoreType.DMA,
        pltpu.SemaphoreType.DMA,
        pltpu.SemaphoreType.DMA,
        pltpu.SemaphoreType.DMA,
        pltpu.SemaphoreType.DMA,
        pltpu.SemaphoreType.DMA,
    ],
)
def _msg_kernel(
    hp_hbm, src_hbm, dst_hbm, z_hbm, acc_out,
    srcall, dstb, rows, acc_s,
    g0, g1, s0, s1, d0, d1,
):
    gs = (g0, g1)
    ss = (s0, s1)
    dsems = (d0, d1)
    c = lax.axis_index("c")
    s = lax.axis_index("s")
    pltpu.sync_copy(z_hbm, rows.at[0])
    pltpu.sync_copy(rows.at[0], acc_s.at[pl.ds(s * ACC_STRIPE, CH)])
    pltpu.sync_copy(
        rows.at[0].at[pl.ds(0, STG2)],
        acc_s.at[pl.ds(s * ACC_STRIPE + CH, STG2)],
    )
    base = s * EPT
    pltpu.sync_copy(src_hbm.at[pl.ds(base, EPT)], srcall)
    plsc.subcore_barrier()
    table = hp_hbm.at[c]

    def fire_chunk(k, i):
        off = pl.multiple_of(i * CH, 8)
        pltpu.async_copy(dst_hbm.at[pl.ds(base + off, CH)], dstb.at[k], dsems[k])
        pltpu.async_copy(table.at[srcall.at[pl.ds(off, CH)]], rows.at[k], gs[k])

    def wait_chunk(k):
        pltpu.make_async_copy(dst_hbm.at[pl.ds(0, CH)], dstb.at[k], dsems[k]).wait()
        pltpu.make_async_copy(
            table.at[srcall.at[pl.ds(0, CH)]], rows.at[k], gs[k]
        ).wait()

    def fire_scatter(k):
        pltpu.async_copy(rows.at[k], acc_s.at[dstb.at[k]], ss[k], add=True)

    def wait_scatter(k):
        pltpu.make_async_copy(rows.at[k], acc_s.at[dstb.at[k]], ss[k]).wait()

    fire_chunk(0, 0)
    fire_chunk(1, 1)

    def round_body(j, carry):
        a = j * 2
        wait_chunk(0)
        fire_scatter(0)
        wait_chunk(1)
        fire_scatter(1)
        wait_scatter(0)
        fire_chunk(0, a + 2)
        wait_scatter(1)
        fire_chunk(1, a + 3)
        return carry

    lax.fori_loop(0, NPAIR - 1, round_body, 0)
    wait_chunk(0)
    fire_scatter(0)
    wait_chunk(1)
    fire_scatter(1)
    wait_scatter(0)
    wait_scatter(1)
    plsc.subcore_barrier()
    pltpu.sync_copy(acc_s.at[pl.ds(s * ACC_STRIPE, CH)], rows.at[0])
    pltpu.sync_copy(rows.at[0], acc_out.at[c, pl.ds(s * ACC_STRIPE, CH)])
    pltpu.sync_copy(
        acc_s.at[pl.ds(s * ACC_STRIPE + CH, STG2)], rows.at[0].at[pl.ds(0, STG2)]
    )
    pltpu.sync_copy(
        rows.at[0].at[pl.ds(0, STG2)],
        acc_out.at[c, pl.ds(s * ACC_STRIPE + CH, STG2)],
    )


BM = 2000


def _dis_body(deg8_ref, dis_ref):
    d = jnp.sum(deg8_ref[...], axis=(0, 3)) + 1.0
    dis_ref[...] = lax.rsqrt(d)


def _dis_call(deg8):
    return pl.pallas_call(
        _dis_body,
        grid=(N // BM,),
        in_specs=[pl.BlockSpec((NC, BM, 3, 8), lambda j: (0, j, 0, 0))],
        out_specs=pl.BlockSpec((BM, 3), lambda j: (j, 0)),
        out_shape=jax.ShapeDtypeStruct((N, 3), jnp.float32),
    )(deg8)


def _split(h):
    return h[:, :DH], h[:, DH:]


def _first_body(x_ref, w_ref, dis_ref, out_ref):
    h = jnp.dot(x_ref[...], w_ref[...], preferred_element_type=jnp.float32)
    h = h * dis_ref[:, 0:1]
    lo, hi = _split(h)
    out_ref[0] = lo
    out_ref[1] = hi


def _first_call(x, W1, dis):
    return pl.pallas_call(
        _first_body,
        grid=(N // BM,),
        in_specs=[
            pl.BlockSpec((BM, D), lambda j: (j, 0)),
            pl.BlockSpec((D, D), lambda j: (0, 0)),
            pl.BlockSpec((BM, 3), lambda j: (j, 0)),
        ],
        out_specs=pl.BlockSpec((NC, BM, DH), lambda j: (0, j, 0)),
        out_shape=jax.ShapeDtypeStruct((NC, N, DH), jnp.float32),
    )(x, W1, dis)


def _mid_body(l, acc_ref, hp_ref, dis_ref, b_ref, w_ref, out_ref):
    t = jnp.concatenate(
        [acc_ref[0] + hp_ref[0], acc_ref[1] + hp_ref[1]], axis=1
    )
    t = t * dis_ref[:, l : l + 1] + b_ref[...]
    t = jnp.maximum(t, 0.0)
    h = jnp.dot(t, w_ref[...], preferred_element_type=jnp.float32)
    h = h * dis_ref[:, l + 1 : l + 2]
    lo, hi = _split(h)
    out_ref[0] = lo
    out_ref[1] = hi


def _mid_call(l, acc, hp, dis, b, Wn):
    return pl.pallas_call(
        functools.partial(_mid_body, l),
        grid=(N // BM,),
        in_specs=[
            pl.BlockSpec((NC, BM, DH), lambda j: (0, j, 0)),
            pl.BlockSpec((NC, BM, DH), lambda j: (0, j, 0)),
            pl.BlockSpec((BM, 3), lambda j: (j, 0)),
            pl.BlockSpec((1, D), lambda j: (0, 0)),
            pl.BlockSpec((D, D), lambda j: (0, 0)),
        ],
        out_specs=pl.BlockSpec((NC, BM, DH), lambda j: (0, j, 0)),
        out_shape=jax.ShapeDtypeStruct((NC, N, DH), jnp.float32),
    )(acc, hp, dis, b, Wn)


def _last_body(acc_ref, hp_ref, dis_ref, b_ref, out_ref):
    t = jnp.concatenate(
        [acc_ref[0] + hp_ref[0], acc_ref[1] + hp_ref[1]], axis=1
    )
    t = t * dis_ref[:, 2:3] + b_ref[...]
    n2 = jnp.sum(t * t, axis=1, keepdims=True)
    out_ref[...] = t * lax.rsqrt(jnp.maximum(n2, 1e-24))


def _last_call(acc, hp, dis, b):
    return pl.pallas_call(
        _last_body,
        grid=(N // BM,),
        in_specs=[
            pl.BlockSpec((NC, BM, DH), lambda j: (0, j, 0)),
            pl.BlockSpec((NC, BM, DH), lambda j: (0, j, 0)),
            pl.BlockSpec((BM, 3), lambda j: (j, 0)),
            pl.BlockSpec((1, D), lambda j: (0, 0)),
        ],
        out_specs=pl.BlockSpec((BM, D), lambda j: (j, 0)),
        out_shape=jax.ShapeDtypeStruct((N, D), jnp.float32),
    )(acc, hp, dis, b)


def kernel(edge_index_list, x, W1, b1, W2, b2, W3, b3):
    src = edge_index_list[:, 0, :]
    dst = edge_index_list[:, 1, :]
    dst_off = (dst * 3 + jnp.arange(3, dtype=jnp.int32)[:, None]).reshape(3 * E)
    eighth = jnp.full((CHD, 8), 0.125, jnp.float32)
    z8 = jnp.zeros((DEG_STRIPE, 8), jnp.float32)
    zrows = jnp.zeros((CH, DH), jnp.float32)

    deg8 = _deg_kernel(dst_off, eighth, z8)[:, : 3 * N].reshape(NC, N, 3, 8)
    dis = _dis_call(deg8)

    hp = _first_call(x, W1, dis)
    acc = _msg_kernel(hp, src[0], dst[0], zrows)
    hp = _mid_call(0, acc, hp, dis, b1.reshape(1, D), W2)
    acc = _msg_kernel(hp, src[1], dst[1], zrows)
    hp = _mid_call(1, acc, hp, dis, b2.reshape(1, D), W3)
    acc = _msg_kernel(hp, src[2], dst[2], zrows)
    return _last_call(acc, hp, dis, b3.reshape(1, D))


# speedup vs baseline: 23.2724x; 1.0043x over previous
import functools

import jax
import jax.numpy as jnp
from jax import lax
from jax.experimental import pallas as pl
from jax.experimental.pallas import tpu as pltpu
from jax.experimental.pallas import tpu_sc as plsc

N = 10000
E = 320000
D = 128
DH = D // 2
NC = 2
NS = 16
NW = NC * NS
EPT = E // NS
CH = 400
NCHUNK = EPT // CH
NPAIR = NCHUNK // 2

ACC_STRIPE = 632
ACC_ROWS = NS * ACC_STRIPE
DEG_STRIPE = 1880
DEG_ROWS = NS * DEG_STRIPE

_mesh = plsc.VectorSubcoreMesh(
    core_axis_name="c", subcore_axis_name="s", num_cores=NC, num_subcores=NS
)
_sc_params = pltpu.CompilerParams(use_tc_tiling_on_sc=False)


CHD = 1200
DEG_EPT = 3 * E // NW
NCHD = DEG_EPT // CHD


@functools.partial(
    pl.kernel,
    out_type=jax.ShapeDtypeStruct((NC, DEG_ROWS, 8), jnp.float32),
    mesh=_mesh,
    compiler_params=_sc_params,
    scratch_types=[
        pltpu.VMEM((DEG_EPT,), jnp.int32),
        pltpu.VMEM((CHD, 8), jnp.float32),
        pltpu.VMEM((DEG_STRIPE, 8), jnp.float32),
        pltpu.VMEM_SHARED((DEG_ROWS, 8), jnp.float32),
        pltpu.SemaphoreType.DMA,
    ],
)
def _deg_kernel(dst_hbm, ones_hbm, z_hbm, deg_out, dstall, onesv, stage, deg_s, ssem):
    c = lax.axis_index("c")
    s = lax.axis_index("s")
    wid = c * NS + s
    pltpu.sync_copy(z_hbm, stage)
    pltpu.sync_copy(stage, deg_s.at[pl.ds(s * DEG_STRIPE, DEG_STRIPE)])
    pltpu.sync_copy(ones_hbm, onesv)
    pltpu.sync_copy(dst_hbm.at[pl.ds(wid * DEG_EPT, DEG_EPT)], dstall)
    plsc.subcore_barrier()

    def fire(i, carry):
        off = pl.multiple_of(i * CHD, 8)
        pltpu.async_copy(onesv, deg_s.at[dstall.at[pl.ds(off, CHD)]], ssem, add=True)
        return carry

    lax.fori_loop(0, NCHD, fire, 0)

    def drain(i, carry):
        pltpu.make_async_copy(onesv, deg_s.at[dstall.at[pl.ds(0, CHD)]], ssem).wait()
        return carry

    lax.fori_loop(0, NCHD, drain, 0)
    plsc.subcore_barrier()
    pltpu.sync_copy(deg_s.at[pl.ds(s * DEG_STRIPE, DEG_STRIPE)], stage)
    pltpu.sync_copy(stage, deg_out.at[c, pl.ds(s * DEG_STRIPE, DEG_STRIPE)])


STG2 = ACC_STRIPE - CH


@functools.partial(
    pl.kernel,
    out_type=jax.ShapeDtypeStruct((NC, ACC_ROWS, DH), jnp.float32),
    mesh=_mesh,
    compiler_params=_sc_params,
    scratch_types=[
        pltpu.VMEM((EPT,), jnp.int32),
        pltpu.VMEM((2, CH), jnp.int32),
        pltpu.VMEM((2, CH, DH), jnp.float32),
        pltpu.VMEM_SHARED((ACC_ROWS, DH), jnp.float32),
        pltpu.SemaphoreType.DMA,
        pltpu.SemaphoreType.DMA,
        pltpu.SemaphoreType.DMA,
        pltpu.SemaphoreType.DMA,
        pltpu.SemaphoreType.DMA,
        pltpu.SemaphoreType.DMA,
    ],
)
def _msg_kernel(
    hp_hbm, src_hbm, dst_hbm, z_hbm, acc_out,
    srcall, dstb, rows, acc_s,
    g0, g1, s0, s1, d0, d1,
):
    gs = (g0, g1)
    ss = (s0, s1)
    dsems = (d0, d1)
    c = lax.axis_index("c")
    s = lax.axis_index("s")
    pltpu.sync_copy(z_hbm, rows.at[0])
    pltpu.sync_copy(rows.at[0], acc_s.at[pl.ds(s * ACC_STRIPE, CH)])
    pltpu.sync_copy(
        rows.at[0].at[pl.ds(0, STG2)],
        acc_s.at[pl.ds(s * ACC_STRIPE + CH, STG2)],
    )
    base = s * EPT
    pltpu.sync_copy(src_hbm.at[pl.ds(base, EPT)], srcall)
    plsc.subcore_barrier()
    table = hp_hbm.at[c]

    def fire_chunk(k, i):
        off = pl.multiple_of(i * CH, 8)
        pltpu.async_copy(dst_hbm.at[pl.ds(base + off, CH)], dstb.at[k], dsems[k])
        pltpu.async_copy(table.at[srcall.at[pl.ds(off, CH)]], rows.at[k], gs[k])

    def wait_chunk(k):
        pltpu.make_async_copy(dst_hbm.at[pl.ds(0, CH)], dstb.at[k], dsems[k]).wait()
        pltpu.make_async_copy(
            table.at[srcall.at[pl.ds(0, CH)]], rows.at[k], gs[k]
        ).wait()

    def fire_scatter(k):
        pltpu.async_copy(rows.at[k], acc_s.at[dstb.at[k]], ss[k], add=True)

    def wait_scatter(k):
        pltpu.make_async_copy(rows.at[k], acc_s.at[dstb.at[k]], ss[k]).wait()

    fire_chunk(0, 0)
    fire_chunk(1, 1)

    def round_body(j, carry):
        a = j * 2
        wait_chunk(0)
        fire_scatter(0)
        wait_chunk(1)
        fire_scatter(1)
        wait_scatter(0)
        fire_chunk(0, a + 2)
        wait_scatter(1)
        fire_chunk(1, a + 3)
        return carry

    lax.fori_loop(0, NPAIR - 1, round_body, 0)
    wait_chunk(0)
    fire_scatter(0)
    wait_chunk(1)
    fire_scatter(1)
    wait_scatter(0)
    wait_scatter(1)
    plsc.subcore_barrier()
    pltpu.sync_copy(acc_s.at[pl.ds(s * ACC_STRIPE, CH)], rows.at[0])
    pltpu.sync_copy(rows.at[0], acc_out.at[c, pl.ds(s * ACC_STRIPE, CH)])
    pltpu.sync_copy(
        acc_s.at[pl.ds(s * ACC_STRIPE + CH, STG2)], rows.at[0].at[pl.ds(0, STG2)]
    )
    pltpu.sync_copy(
        rows.at[0].at[pl.ds(0, STG2)],
        acc_out.at[c, pl.ds(s * ACC_STRIPE + CH, STG2)],
    )


BM = 2000


def _split(h):
    return h[:, :DH], h[:, DH:]


def _mm1_body(x_ref, w_ref, out_ref):
    out_ref[...] = jnp.dot(
        x_ref[...], w_ref[...], preferred_element_type=jnp.float32
    )


def _mm1_call(x, W1):
    return pl.pallas_call(
        _mm1_body,
        grid=(N // BM,),
        in_specs=[
            pl.BlockSpec((BM, D), lambda j: (j, 0)),
            pl.BlockSpec((D, D), lambda j: (0, 0)),
        ],
        out_specs=pl.BlockSpec((BM, D), lambda j: (j, 0)),
        out_shape=jax.ShapeDtypeStruct((N, D), jnp.float32),
    )(x, W1)


def _disscale_body(deg8_ref, h1_ref, dis_ref, hp_ref):
    d = jnp.sum(deg8_ref[...], axis=(0, 3)) + 1.0
    dis = lax.rsqrt(d)
    dis_ref[...] = dis
    h = h1_ref[...] * dis[:, 0:1]
    lo, hi = _split(h)
    hp_ref[0] = lo
    hp_ref[1] = hi


def _disscale_call(deg8, h1):
    return pl.pallas_call(
        _disscale_body,
        grid=(N // BM,),
        in_specs=[
            pl.BlockSpec((NC, BM, 3, 8), lambda j: (0, j, 0, 0)),
            pl.BlockSpec((BM, D), lambda j: (j, 0)),
        ],
        out_specs=[
            pl.BlockSpec((BM, 3), lambda j: (j, 0)),
            pl.BlockSpec((NC, BM, DH), lambda j: (0, j, 0)),
        ],
        out_shape=[
            jax.ShapeDtypeStruct((N, 3), jnp.float32),
            jax.ShapeDtypeStruct((NC, N, DH), jnp.float32),
        ],
    )(deg8, h1)


def _mid_body(l, acc_ref, hp_ref, dis_ref, b_ref, w_ref, out_ref):
    t = jnp.concatenate(
        [acc_ref[0] + hp_ref[0], acc_ref[1] + hp_ref[1]], axis=1
    )
    t = t * dis_ref[:, l : l + 1] + b_ref[...]
    t = jnp.maximum(t, 0.0)
    h = jnp.dot(t, w_ref[...], preferred_element_type=jnp.float32)
    h = h * dis_ref[:, l + 1 : l + 2]
    lo, hi = _split(h)
    out_ref[0] = lo
    out_ref[1] = hi


def _mid_call(l, acc, hp, dis, b, Wn):
    return pl.pallas_call(
        functools.partial(_mid_body, l),
        grid=(N // BM,),
        in_specs=[
            pl.BlockSpec((NC, BM, DH), lambda j: (0, j, 0)),
            pl.BlockSpec((NC, BM, DH), lambda j: (0, j, 0)),
            pl.BlockSpec((BM, 3), lambda j: (j, 0)),
            pl.BlockSpec((1, D), lambda j: (0, 0)),
            pl.BlockSpec((D, D), lambda j: (0, 0)),
        ],
        out_specs=pl.BlockSpec((NC, BM, DH), lambda j: (0, j, 0)),
        out_shape=jax.ShapeDtypeStruct((NC, N, DH), jnp.float32),
    )(acc, hp, dis, b, Wn)


def _last_body(acc_ref, hp_ref, dis_ref, b_ref, out_ref):
    t = jnp.concatenate(
        [acc_ref[0] + hp_ref[0], acc_ref[1] + hp_ref[1]], axis=1
    )
    t = t * dis_ref[:, 2:3] + b_ref[...]
    n2 = jnp.sum(t * t, axis=1, keepdims=True)
    out_ref[...] = t * lax.rsqrt(jnp.maximum(n2, 1e-24))


def _last_call(acc, hp, dis, b):
    return pl.pallas_call(
        _last_body,
        grid=(N // BM,),
        in_specs=[
            pl.BlockSpec((NC, BM, DH), lambda j: (0, j, 0)),
            pl.BlockSpec((NC, BM, DH), lambda j: (0, j, 0)),
            pl.BlockSpec((BM, 3), lambda j: (j, 0)),
            pl.BlockSpec((1, D), lambda j: (0, 0)),
        ],
        out_specs=pl.BlockSpec((BM, D), lambda j: (j, 0)),
        out_shape=jax.ShapeDtypeStruct((N, D), jnp.float32),
    )(acc, hp, dis, b)


def kernel(edge_index_list, x, W1, b1, W2, b2, W3, b3):
    src = edge_index_list[:, 0, :]
    dst = edge_index_list[:, 1, :]
    dst_off = (dst * 3 + jnp.arange(3, dtype=jnp.int32)[:, None]).reshape(3 * E)
    eighth = jnp.full((CHD, 8), 0.125, jnp.float32)
    z8 = jnp.zeros((DEG_STRIPE, 8), jnp.float32)
    zrows = jnp.zeros((CH, DH), jnp.float32)

    h1 = _mm1_call(x, W1)
    deg8 = _deg_kernel(dst_off, eighth, z8)[:, : 3 * N].reshape(NC, N, 3, 8)
    dis, hp = _disscale_call(deg8, h1)
    acc = _msg_kernel(hp, src[0], dst[0], zrows)
    hp = _mid_call(0, acc, hp, dis, b1.reshape(1, D), W2)
    acc = _msg_kernel(hp, src[1], dst[1], zrows)
    hp = _mid_call(1, acc, hp, dis, b2.reshape(1, D), W3)
    acc = _msg_kernel(hp, src[2], dst[2], zrows)
    return _last_call(acc, hp, dis, b3.reshape(1, D))


# speedup vs baseline: 31.0237x; 1.3331x over previous
import functools

import jax
import jax.numpy as jnp
from jax import lax
from jax.experimental import pallas as pl
from jax.experimental.pallas import tpu as pltpu
from jax.experimental.pallas import tpu_sc as plsc

N = 10000
E = 320000
D = 128
DH = D // 2
NC = 2
NS = 16
NW = NC * NS
EPT = E // NS
CH = 400
NCHUNK = EPT // CH
NPAIR = NCHUNK // 2

ACC_STRIPE = 632
ACC_ROWS = NS * ACC_STRIPE

_mesh = plsc.VectorSubcoreMesh(
    core_axis_name="c", subcore_axis_name="s", num_cores=NC, num_subcores=NS
)
_sc_params = pltpu.CompilerParams(use_tc_tiling_on_sc=False)


CHD = 2000
DEG_LPT = E // NW
NCHD = DEG_LPT // CHD
DEG_PAD = 10240
DEG_STR = DEG_PAD // NS


@functools.partial(
    pl.kernel,
    out_type=jax.ShapeDtypeStruct((NC, 3, DEG_PAD, 8), jnp.float32),
    mesh=_mesh,
    compiler_params=_sc_params,
    scratch_types=[
        pltpu.VMEM((3 * DEG_LPT,), jnp.int32),
        pltpu.VMEM((CHD, 8), jnp.float32),
        pltpu.VMEM((DEG_STR, 8), jnp.float32),
        pltpu.VMEM_SHARED((DEG_PAD, 8), jnp.float32),
        pltpu.VMEM_SHARED((DEG_PAD, 8), jnp.float32),
        pltpu.VMEM_SHARED((DEG_PAD, 8), jnp.float32),
        pltpu.SemaphoreType.DMA,
    ],
)
def _deg_kernel(d0_hbm, d1_hbm, d2_hbm, ones_hbm, z_hbm, deg_out, dstall, onesv, stage, t0, t1, t2, ssem):
    tables = (t0, t1, t2)
    dsts = (d0_hbm, d1_hbm, d2_hbm)
    c = lax.axis_index("c")
    s = lax.axis_index("s")
    wid = c * NS + s
    pltpu.sync_copy(z_hbm, stage)
    for l in range(3):
        pltpu.sync_copy(stage, tables[l].at[pl.ds(s * DEG_STR, DEG_STR)])
        pltpu.sync_copy(
            dsts[l].at[pl.ds(wid * DEG_LPT, DEG_LPT)],
            dstall.at[pl.ds(l * DEG_LPT, DEG_LPT)],
        )
    pltpu.sync_copy(ones_hbm, onesv)
    plsc.subcore_barrier()

    for l in range(3):
        def fire(i, carry, l=l):
            off = pl.multiple_of(l * DEG_LPT + i * CHD, 8)
            pltpu.async_copy(
                onesv, tables[l].at[dstall.at[pl.ds(off, CHD)]], ssem, add=True
            )
            return carry

        lax.fori_loop(0, NCHD, fire, 0)

    def drain(i, carry):
        pltpu.make_async_copy(onesv, t0.at[dstall.at[pl.ds(0, CHD)]], ssem).wait()
        return carry

    lax.fori_loop(0, 3 * NCHD, drain, 0)
    plsc.subcore_barrier()
    for l in range(3):
        pltpu.sync_copy(tables[l].at[pl.ds(s * DEG_STR, DEG_STR)], stage)
        pltpu.sync_copy(stage, deg_out.at[c, l, pl.ds(s * DEG_STR, DEG_STR)])


STG2 = ACC_STRIPE - CH


@functools.partial(
    pl.kernel,
    out_type=jax.ShapeDtypeStruct((NC, ACC_ROWS, DH), jnp.float32),
    mesh=_mesh,
    compiler_params=_sc_params,
    scratch_types=[
        pltpu.VMEM((EPT,), jnp.int32),
        pltpu.VMEM((2, CH), jnp.int32),
        pltpu.VMEM((2, CH, DH), jnp.float32),
        pltpu.VMEM_SHARED((ACC_ROWS, DH), jnp.float32),
        pltpu.SemaphoreType.DMA,
        pltpu.SemaphoreType.DMA,
        pltpu.SemaphoreType.DMA,
        pltpu.SemaphoreType.DMA,
        pltpu.SemaphoreType.DMA,
        pltpu.SemaphoreType.DMA,
    ],
)
def _msg_kernel(
    hp_hbm, src_hbm, dst_hbm, z_hbm, acc_out,
    srcall, dstb, rows, acc_s,
    g0, g1, s0, s1, d0, d1,
):
    gs = (g0, g1)
    ss = (s0, s1)
    dsems = (d0, d1)
    c = lax.axis_index("c")
    s = lax.axis_index("s")
    pltpu.sync_copy(z_hbm, rows.at[0])
    pltpu.sync_copy(rows.at[0], acc_s.at[pl.ds(s * ACC_STRIPE, CH)])
    pltpu.sync_copy(
        rows.at[0].at[pl.ds(0, STG2)],
        acc_s.at[pl.ds(s * ACC_STRIPE + CH, STG2)],
    )
    base = s * EPT
    pltpu.sync_copy(src_hbm.at[pl.ds(base, EPT)], srcall)
    plsc.subcore_barrier()
    table = hp_hbm.at[c]

    def fire_chunk(k, i):
        off = pl.multiple_of(i * CH, 8)
        pltpu.async_copy(
            dst_hbm.at[pl.ds(base + off, CH)], dstb.at[k], dsems[k]
        )
        pltpu.async_copy(table.at[srcall.at[pl.ds(off, CH)]], rows.at[k], gs[k])

    def wait_chunk(k):
        pltpu.make_async_copy(
            dst_hbm.at[pl.ds(0, CH)], dstb.at[k], dsems[k]
        ).wait()
        pltpu.make_async_copy(
            table.at[srcall.at[pl.ds(0, CH)]], rows.at[k], gs[k]
        ).wait()

    def fire_scatter(k):
        pltpu.async_copy(rows.at[k], acc_s.at[dstb.at[k]], ss[k], add=True)

    def wait_scatter(k):
        pltpu.make_async_copy(rows.at[k], acc_s.at[dstb.at[k]], ss[k]).wait()

    fire_chunk(0, 0)
    fire_chunk(1, 1)

    def round_body(j, carry):
        a = j * 2
        wait_chunk(0)
        fire_scatter(0)
        wait_chunk(1)
        fire_scatter(1)
        wait_scatter(0)
        fire_chunk(0, a + 2)
        wait_scatter(1)
        fire_chunk(1, a + 3)
        return carry

    lax.fori_loop(0, NPAIR - 1, round_body, 0)
    wait_chunk(0)
    fire_scatter(0)
    wait_chunk(1)
    fire_scatter(1)
    wait_scatter(0)
    wait_scatter(1)
    plsc.subcore_barrier()
    pltpu.sync_copy(acc_s.at[pl.ds(s * ACC_STRIPE, CH)], rows.at[0])
    pltpu.sync_copy(rows.at[0], acc_out.at[c, pl.ds(s * ACC_STRIPE, CH)])
    pltpu.sync_copy(
        acc_s.at[pl.ds(s * ACC_STRIPE + CH, STG2)],
        rows.at[0].at[pl.ds(0, STG2)],
    )
    pltpu.sync_copy(
        rows.at[0].at[pl.ds(0, STG2)],
        acc_out.at[c, pl.ds(s * ACC_STRIPE + CH, STG2)],
    )


def _repack_body(eil_ref, s0, d0, s1, d1, s2, d2):
    pid = pl.program_id(0)
    outs = ((s0, d0), (s1, d1), (s2, d2))
    for l in range(3):
        @pl.when(pid == l)
        def _(l=l):
            outs[l][0][...] = eil_ref[0, 0, :]
            outs[l][1][...] = eil_ref[0, 1, :]


def _repack_call(eil):
    flat = jax.ShapeDtypeStruct((E,), jnp.int32)
    return pl.pallas_call(
        _repack_body,
        grid=(3,),
        in_specs=[pl.BlockSpec((1, 2, E), lambda l: (l, 0, 0))],
        out_specs=[pl.BlockSpec((E,), lambda l: (0,))] * 6,
        out_shape=[flat] * 6,
    )(eil)


BM = 2000


def _split(h):
    return h[:, :DH], h[:, DH:]


def _mm1_body(x_ref, w_ref, out_ref):
    out_ref[...] = jnp.dot(
        x_ref[...], w_ref[...], preferred_element_type=jnp.float32
    )


def _mm1_call(x, W1):
    return pl.pallas_call(
        _mm1_body,
        grid=(N // BM,),
        in_specs=[
            pl.BlockSpec((BM, D), lambda j: (j, 0)),
            pl.BlockSpec((D, D), lambda j: (0, 0)),
        ],
        out_specs=pl.BlockSpec((BM, D), lambda j: (j, 0)),
        out_shape=jax.ShapeDtypeStruct((N, D), jnp.float32),
    )(x, W1)


def _disscale_body(deg8_ref, h1_ref, dis_ref, hp_ref):
    d = jnp.sum(deg8_ref[...], axis=(0, 3)) + 1.0
    dis_t = lax.rsqrt(d)
    dis_ref[...] = dis_t.T
    h = h1_ref[...] * dis_t[0:1, :].T
    lo, hi = _split(h)
    hp_ref[0] = lo
    hp_ref[1] = hi


def _disscale_call(deg8, h1):
    return pl.pallas_call(
        _disscale_body,
        grid=(N // BM,),
        in_specs=[
            pl.BlockSpec((NC, 3, BM, 8), lambda j: (0, 0, j, 0)),
            pl.BlockSpec((BM, D), lambda j: (j, 0)),
        ],
        out_specs=[
            pl.BlockSpec((BM, 3), lambda j: (j, 0)),
            pl.BlockSpec((NC, BM, DH), lambda j: (0, j, 0)),
        ],
        out_shape=[
            jax.ShapeDtypeStruct((N, 3), jnp.float32),
            jax.ShapeDtypeStruct((NC, N, DH), jnp.float32),
        ],
    )(deg8, h1)


def _mid_body(l, acc_ref, hp_ref, dis_ref, b_ref, w_ref, out_ref):
    t = jnp.concatenate(
        [acc_ref[0] + hp_ref[0], acc_ref[1] + hp_ref[1]], axis=1
    )
    t = t * dis_ref[:, l : l + 1] + b_ref[...]
    t = jnp.maximum(t, 0.0)
    h = jnp.dot(t, w_ref[...], preferred_element_type=jnp.float32)
    h = h * dis_ref[:, l + 1 : l + 2]
    lo, hi = _split(h)
    out_ref[0] = lo
    out_ref[1] = hi


def _mid_call(l, acc, hp, dis, b, Wn):
    return pl.pallas_call(
        functools.partial(_mid_body, l),
        grid=(N // BM,),
        in_specs=[
            pl.BlockSpec((NC, BM, DH), lambda j: (0, j, 0)),
            pl.BlockSpec((NC, BM, DH), lambda j: (0, j, 0)),
            pl.BlockSpec((BM, 3), lambda j: (j, 0)),
            pl.BlockSpec((1, D), lambda j: (0, 0)),
            pl.BlockSpec((D, D), lambda j: (0, 0)),
        ],
        out_specs=pl.BlockSpec((NC, BM, DH), lambda j: (0, j, 0)),
        out_shape=jax.ShapeDtypeStruct((NC, N, DH), jnp.float32),
    )(acc, hp, dis, b, Wn)


def _last_body(acc_ref, hp_ref, dis_ref, b_ref, out_ref):
    t = jnp.concatenate(
        [acc_ref[0] + hp_ref[0], acc_ref[1] + hp_ref[1]], axis=1
    )
    t = t * dis_ref[:, 2:3] + b_ref[...]
    n2 = jnp.sum(t * t, axis=1, keepdims=True)
    out_ref[...] = t * lax.rsqrt(jnp.maximum(n2, 1e-24))


def _last_call(acc, hp, dis, b):
    return pl.pallas_call(
        _last_body,
        grid=(N // BM,),
        in_specs=[
            pl.BlockSpec((NC, BM, DH), lambda j: (0, j, 0)),
            pl.BlockSpec((NC, BM, DH), lambda j: (0, j, 0)),
            pl.BlockSpec((BM, 3), lambda j: (j, 0)),
            pl.BlockSpec((1, D), lambda j: (0, 0)),
        ],
        out_specs=pl.BlockSpec((BM, D), lambda j: (j, 0)),
        out_shape=jax.ShapeDtypeStruct((N, D), jnp.float32),
    )(acc, hp, dis, b)


def kernel(edge_index_list, x, W1, b1, W2, b2, W3, b3):
    s0, d0, s1, d1, s2, d2 = _repack_call(edge_index_list)
    eighth = jnp.full((CHD, 8), 0.125, jnp.float32)
    z8 = jnp.zeros((DEG_STR, 8), jnp.float32)
    zrows = jnp.zeros((CH, DH), jnp.float32)

    h1 = _mm1_call(x, W1)
    deg8 = _deg_kernel(d0, d1, d2, eighth, z8)
    dis, hp = _disscale_call(deg8, h1)
    acc = _msg_kernel(hp, s0, d0, zrows)
    hp = _mid_call(0, acc, hp, dis, b1.reshape(1, D), W2)
    acc = _msg_kernel(hp, s1, d1, zrows)
    hp = _mid_call(1, acc, hp, dis, b2.reshape(1, D), W3)
    acc = _msg_kernel(hp, s2, d2, zrows)
    return _last_call(acc, hp, dis, b3.reshape(1, D))


# speedup vs baseline: 34.8565x; 1.1235x over previous
import functools

import jax
import jax.numpy as jnp
from jax import lax
from jax.experimental import pallas as pl
from jax.experimental.pallas import tpu as pltpu
from jax.experimental.pallas import tpu_sc as plsc

N = 10000
E = 320000
D = 128
DH = D // 2
NC = 2
NS = 16
NW = NC * NS
EPT = E // NS
CH = 400
NCHUNK = EPT // CH
KBUF = 3
NFULL = NCHUNK // KBUF - 1
NTAIL = NCHUNK - KBUF * (NFULL + 1)

ACC_STRIPE = 632
ACC_ROWS = NS * ACC_STRIPE

_mesh = plsc.VectorSubcoreMesh(
    core_axis_name="c", subcore_axis_name="s", num_cores=NC, num_subcores=NS
)
_sc_params = pltpu.CompilerParams(use_tc_tiling_on_sc=False)


CHD = 2000
DEG_LPT = E // NW
NCHD = DEG_LPT // CHD
DEG_PAD = 10240
DEG_STR = DEG_PAD // NS


@functools.partial(
    pl.kernel,
    out_type=jax.ShapeDtypeStruct((NC, 3, DEG_PAD, 8), jnp.float32),
    mesh=_mesh,
    compiler_params=_sc_params,
    scratch_types=[
        pltpu.VMEM((3 * DEG_LPT,), jnp.int32),
        pltpu.VMEM((CHD, 8), jnp.float32),
        pltpu.VMEM((DEG_STR, 8), jnp.float32),
        pltpu.VMEM_SHARED((DEG_PAD, 8), jnp.float32),
        pltpu.VMEM_SHARED((DEG_PAD, 8), jnp.float32),
        pltpu.VMEM_SHARED((DEG_PAD, 8), jnp.float32),
        pltpu.SemaphoreType.DMA,
    ],
)
def _deg_kernel(d0_hbm, d1_hbm, d2_hbm, ones_hbm, z_hbm, deg_out, dstall, onesv, stage, t0, t1, t2, ssem):
    tables = (t0, t1, t2)
    dsts = (d0_hbm, d1_hbm, d2_hbm)
    c = lax.axis_index("c")
    s = lax.axis_index("s")
    wid = c * NS + s
    pltpu.sync_copy(z_hbm, stage)
    for l in range(3):
        pltpu.sync_copy(stage, tables[l].at[pl.ds(s * DEG_STR, DEG_STR)])
        pltpu.sync_copy(
            dsts[l].at[pl.ds(wid * DEG_LPT, DEG_LPT)],
            dstall.at[pl.ds(l * DEG_LPT, DEG_LPT)],
        )
    pltpu.sync_copy(ones_hbm, onesv)
    plsc.subcore_barrier()

    for l in range(3):
        def fire(i, carry, l=l):
            off = pl.multiple_of(l * DEG_LPT + i * CHD, 8)
            pltpu.async_copy(
                onesv, tables[l].at[dstall.at[pl.ds(off, CHD)]], ssem, add=True
            )
            return carry

        lax.fori_loop(0, NCHD, fire, 0)

    def drain(i, carry):
        pltpu.make_async_copy(onesv, t0.at[dstall.at[pl.ds(0, CHD)]], ssem).wait()
        return carry

    lax.fori_loop(0, 3 * NCHD, drain, 0)
    plsc.subcore_barrier()
    for l in range(3):
        pltpu.sync_copy(tables[l].at[pl.ds(s * DEG_STR, DEG_STR)], stage)
        pltpu.sync_copy(stage, deg_out.at[c, l, pl.ds(s * DEG_STR, DEG_STR)])


STG2 = ACC_STRIPE - CH


@functools.partial(
    pl.kernel,
    out_type=jax.ShapeDtypeStruct((NC, ACC_ROWS, DH), jnp.float32),
    mesh=_mesh,
    compiler_params=_sc_params,
    scratch_types=[
        pltpu.VMEM((KBUF, CH), jnp.int32),
        pltpu.VMEM((KBUF, CH), jnp.int32),
        pltpu.VMEM((KBUF, CH, DH), jnp.float32),
        pltpu.VMEM_SHARED((ACC_ROWS, DH), jnp.float32),
        pltpu.SemaphoreType.DMA,
        pltpu.SemaphoreType.DMA,
        pltpu.SemaphoreType.DMA,
        pltpu.SemaphoreType.DMA,
        pltpu.SemaphoreType.DMA,
        pltpu.SemaphoreType.DMA,
        pltpu.SemaphoreType.DMA,
        pltpu.SemaphoreType.DMA,
        pltpu.SemaphoreType.DMA,
        pltpu.SemaphoreType.DMA,
        pltpu.SemaphoreType.DMA,
        pltpu.SemaphoreType.DMA,
    ],
)
def _msg_kernel(
    hp_hbm, src_hbm, dst_hbm, z_hbm, acc_out,
    srcb, dstb, rows, acc_s,
    l0, l1, l2, g0, g1, g2, s0, s1, s2, d0, d1, d2,
):
    ls = (l0, l1, l2)
    gs = (g0, g1, g2)
    ss = (s0, s1, s2)
    dsems = (d0, d1, d2)
    c = lax.axis_index("c")
    s = lax.axis_index("s")
    pltpu.sync_copy(z_hbm, rows.at[0])
    pltpu.sync_copy(rows.at[0], acc_s.at[pl.ds(s * ACC_STRIPE, CH)])
    pltpu.sync_copy(
        rows.at[0].at[pl.ds(0, STG2)],
        acc_s.at[pl.ds(s * ACC_STRIPE + CH, STG2)],
    )
    plsc.subcore_barrier()
    base = s * EPT
    table = hp_hbm.at[c]

    def fire_load(k, i):
        off = pl.multiple_of(i * CH, 8)
        pltpu.async_copy(src_hbm.at[pl.ds(base + off, CH)], srcb.at[k], ls[k])
        pltpu.async_copy(dst_hbm.at[pl.ds(base + off, CH)], dstb.at[k], dsems[k])

    def fire_gather(k):
        pltpu.make_async_copy(src_hbm.at[pl.ds(0, CH)], srcb.at[k], ls[k]).wait()
        pltpu.async_copy(table.at[srcb.at[k]], rows.at[k], gs[k])

    def fire_scatter(k):
        pltpu.make_async_copy(dst_hbm.at[pl.ds(0, CH)], dstb.at[k], dsems[k]).wait()
        pltpu.make_async_copy(table.at[srcb.at[k]], rows.at[k], gs[k]).wait()
        pltpu.async_copy(rows.at[k], acc_s.at[dstb.at[k]], ss[k], add=True)

    def wait_scatter(k):
        pltpu.make_async_copy(rows.at[k], acc_s.at[dstb.at[k]], ss[k]).wait()

    for k in range(KBUF):
        fire_load(k, k)
    for k in range(KBUF):
        fire_gather(k)

    def round_body(j, carry):
        a = j * KBUF
        for k in range(KBUF):
            fire_scatter(k)
        for k in range(KBUF):
            wait_scatter(k)
            fire_load(k, a + KBUF + k)
            fire_gather(k)
        return carry

    lax.fori_loop(0, NFULL, round_body, 0)
    tb = (NFULL + 1) * KBUF
    for k in range(KBUF):
        fire_scatter(k)
    for k in range(KBUF):
        wait_scatter(k)
        if k < NTAIL:
            fire_load(k, tb + k)
            fire_gather(k)
    for k in range(NTAIL):
        fire_scatter(k)
    for k in range(NTAIL):
        wait_scatter(k)
    plsc.subcore_barrier()
    pltpu.sync_copy(acc_s.at[pl.ds(s * ACC_STRIPE, CH)], rows.at[0])
    pltpu.sync_copy(rows.at[0], acc_out.at[c, pl.ds(s * ACC_STRIPE, CH)])
    pltpu.sync_copy(
        acc_s.at[pl.ds(s * ACC_STRIPE + CH, STG2)],
        rows.at[0].at[pl.ds(0, STG2)],
    )
    pltpu.sync_copy(
        rows.at[0].at[pl.ds(0, STG2)],
        acc_out.at[c, pl.ds(s * ACC_STRIPE + CH, STG2)],
    )


def _repack_body(eil_ref, s0, d0, s1, d1, s2, d2):
    pid = pl.program_id(0)
    outs = ((s0, d0), (s1, d1), (s2, d2))
    for l in range(3):
        @pl.when(pid == l)
        def _(l=l):
            outs[l][0][...] = eil_ref[0, 0, :]
            outs[l][1][...] = eil_ref[0, 1, :]


def _repack_call(eil):
    flat = jax.ShapeDtypeStruct((E,), jnp.int32)
    return pl.pallas_call(
        _repack_body,
        grid=(3,),
        in_specs=[pl.BlockSpec((1, 2, E), lambda l: (l, 0, 0))],
        out_specs=[pl.BlockSpec((E,), lambda l: (0,))] * 6,
        out_shape=[flat] * 6,
    )(eil)


BM = 2000


def _split(h):
    return h[:, :DH], h[:, DH:]


def _mm1_body(x_ref, w_ref, out_ref):
    out_ref[...] = jnp.dot(
        x_ref[...], w_ref[...], preferred_element_type=jnp.float32
    )


def _mm1_call(x, W1):
    return pl.pallas_call(
        _mm1_body,
        grid=(N // BM,),
        in_specs=[
            pl.BlockSpec((BM, D), lambda j: (j, 0)),
            pl.BlockSpec((D, D), lambda j: (0, 0)),
        ],
        out_specs=pl.BlockSpec((BM, D), lambda j: (j, 0)),
        out_shape=jax.ShapeDtypeStruct((N, D), jnp.float32),
    )(x, W1)


def _disscale_body(deg8_ref, h1_ref, dis_ref, hp_ref):
    d = jnp.sum(deg8_ref[...], axis=(0, 3)) + 1.0
    dis_t = lax.rsqrt(d)
    dis_ref[...] = dis_t.T
    h = h1_ref[...] * dis_t[0:1, :].T
    lo, hi = _split(h)
    hp_ref[0] = lo
    hp_ref[1] = hi


def _disscale_call(deg8, h1):
    return pl.pallas_call(
        _disscale_body,
        grid=(N // BM,),
        in_specs=[
            pl.BlockSpec((NC, 3, BM, 8), lambda j: (0, 0, j, 0)),
            pl.BlockSpec((BM, D), lambda j: (j, 0)),
        ],
        out_specs=[
            pl.BlockSpec((BM, 3), lambda j: (j, 0)),
            pl.BlockSpec((NC, BM, DH), lambda j: (0, j, 0)),
        ],
        out_shape=[
            jax.ShapeDtypeStruct((N, 3), jnp.float32),
            jax.ShapeDtypeStruct((NC, N, DH), jnp.float32),
        ],
    )(deg8, h1)


def _mid_body(l, acc_ref, hp_ref, dis_ref, b_ref, w_ref, out_ref):
    t = jnp.concatenate(
        [acc_ref[0] + hp_ref[0], acc_ref[1] + hp_ref[1]], axis=1
    )
    t = t * dis_ref[:, l : l + 1] + b_ref[...]
    t = jnp.maximum(t, 0.0)
    h = jnp.dot(t, w_ref[...], preferred_element_type=jnp.float32)
    h = h * dis_ref[:, l + 1 : l + 2]
    lo, hi = _split(h)
    out_ref[0] = lo
    out_ref[1] = hi


def _mid_call(l, acc, hp, dis, b, Wn):
    return pl.pallas_call(
        functools.partial(_mid_body, l),
        grid=(N // BM,),
        in_specs=[
            pl.BlockSpec((NC, BM, DH), lambda j: (0, j, 0)),
            pl.BlockSpec((NC, BM, DH), lambda j: (0, j, 0)),
            pl.BlockSpec((BM, 3), lambda j: (j, 0)),
            pl.BlockSpec((1, D), lambda j: (0, 0)),
            pl.BlockSpec((D, D), lambda j: (0, 0)),
        ],
        out_specs=pl.BlockSpec((NC, BM, DH), lambda j: (0, j, 0)),
        out_shape=jax.ShapeDtypeStruct((NC, N, DH), jnp.float32),
    )(acc, hp, dis, b, Wn)


def _last_body(acc_ref, hp_ref, dis_ref, b_ref, out_ref):
    t = jnp.concatenate(
        [acc_ref[0] + hp_ref[0], acc_ref[1] + hp_ref[1]], axis=1
    )
    t = t * dis_ref[:, 2:3] + b_ref[...]
    n2 = jnp.sum(t * t, axis=1, keepdims=True)
    out_ref[...] = t * lax.rsqrt(jnp.maximum(n2, 1e-24))


def _last_call(acc, hp, dis, b):
    return pl.pallas_call(
        _last_body,
        grid=(N // BM,),
        in_specs=[
            pl.BlockSpec((NC, BM, DH), lambda j: (0, j, 0)),
            pl.BlockSpec((NC, BM, DH), lambda j: (0, j, 0)),
            pl.BlockSpec((BM, 3), lambda j: (j, 0)),
            pl.BlockSpec((1, D), lambda j: (0, 0)),
        ],
        out_specs=pl.BlockSpec((BM, D), lambda j: (j, 0)),
        out_shape=jax.ShapeDtypeStruct((N, D), jnp.float32),
    )(acc, hp, dis, b)


def kernel(edge_index_list, x, W1, b1, W2, b2, W3, b3):
    s0, d0, s1, d1, s2, d2 = _repack_call(edge_index_list)
    eighth = jnp.full((CHD, 8), 0.125, jnp.float32)
    z8 = jnp.zeros((DEG_STR, 8), jnp.float32)
    zrows = jnp.zeros((CH, DH), jnp.float32)

    h1 = _mm1_call(x, W1)
    deg8 = _deg_kernel(d0, d1, d2, eighth, z8)
    dis, hp = _disscale_call(deg8, h1)
    acc = _msg_kernel(hp, s0, d0, zrows)
    hp = _mid_call(0, acc, hp, dis, b1.reshape(1, D), W2)
    acc = _msg_kernel(hp, s1, d1, zrows)
    hp = _mid_call(1, acc, hp, dis, b2.reshape(1, D), W3)
    acc = _msg_kernel(hp, s2, d2, zrows)
    return _last_call(acc, hp, dis, b3.reshape(1, D))


# speedup vs baseline: 37.5134x; 1.0762x over previous
import functools

import jax
import jax.numpy as jnp
from jax import lax
from jax.experimental import pallas as pl
from jax.experimental.pallas import tpu as pltpu
from jax.experimental.pallas import tpu_sc as plsc

N = 10000
E = 320000
D = 128
DH = D // 2
NC = 2
NS = 16
NW = NC * NS
EPT = E // NS
CH = 400
NCHUNK = EPT // CH
KBUF = 3
NFULL = NCHUNK // KBUF - 1
NTAIL = NCHUNK - KBUF * (NFULL + 1)

ACC_STRIPE = 632
ACC_ROWS = NS * ACC_STRIPE

_mesh = plsc.VectorSubcoreMesh(
    core_axis_name="c", subcore_axis_name="s", num_cores=NC, num_subcores=NS
)
_sc_params = pltpu.CompilerParams(use_tc_tiling_on_sc=False)


CHD = 2000
DEG_LPT = E // NW
NCHD = DEG_LPT // CHD
DEG_PAD = 10240
DEG_STR = DEG_PAD // NS


@functools.partial(
    pl.kernel,
    out_type=jax.ShapeDtypeStruct((NC, 3, DEG_PAD, 8), jnp.float32),
    mesh=_mesh,
    compiler_params=_sc_params,
    scratch_types=[
        pltpu.VMEM((3 * DEG_LPT,), jnp.int32),
        pltpu.VMEM((CHD, 8), jnp.float32),
        pltpu.VMEM((DEG_STR, 8), jnp.float32),
        pltpu.VMEM_SHARED((DEG_PAD, 8), jnp.float32),
        pltpu.VMEM_SHARED((DEG_PAD, 8), jnp.float32),
        pltpu.VMEM_SHARED((DEG_PAD, 8), jnp.float32),
        pltpu.SemaphoreType.DMA,
    ],
)
def _deg_kernel(d0_hbm, d1_hbm, d2_hbm, ones_hbm, z_hbm, deg_out, dstall, onesv, stage, t0, t1, t2, ssem):
    tables = (t0, t1, t2)
    dsts = (d0_hbm, d1_hbm, d2_hbm)
    c = lax.axis_index("c")
    s = lax.axis_index("s")
    wid = c * NS + s
    pltpu.sync_copy(z_hbm, stage)
    for l in range(3):
        pltpu.sync_copy(stage, tables[l].at[pl.ds(s * DEG_STR, DEG_STR)])
        pltpu.sync_copy(
            dsts[l].at[pl.ds(wid * DEG_LPT, DEG_LPT)],
            dstall.at[pl.ds(l * DEG_LPT, DEG_LPT)],
        )
    pltpu.sync_copy(ones_hbm, onesv)
    plsc.subcore_barrier()

    for l in range(3):
        def fire(i, carry, l=l):
            off = pl.multiple_of(l * DEG_LPT + i * CHD, 8)
            pltpu.async_copy(
                onesv, tables[l].at[dstall.at[pl.ds(off, CHD)]], ssem, add=True
            )
            return carry

        lax.fori_loop(0, NCHD, fire, 0)

    def drain(i, carry):
        pltpu.make_async_copy(onesv, t0.at[dstall.at[pl.ds(0, CHD)]], ssem).wait()
        return carry

    lax.fori_loop(0, 3 * NCHD, drain, 0)
    plsc.subcore_barrier()
    for l in range(3):
        pltpu.sync_copy(tables[l].at[pl.ds(s * DEG_STR, DEG_STR)], stage)
        pltpu.sync_copy(stage, deg_out.at[c, l, pl.ds(s * DEG_STR, DEG_STR)])


STG2 = ACC_STRIPE - CH


@functools.partial(
    pl.kernel,
    out_type=jax.ShapeDtypeStruct((NC, ACC_ROWS, DH), jnp.float32),
    mesh=_mesh,
    compiler_params=_sc_params,
    scratch_types=[
        pltpu.VMEM((KBUF, CH), jnp.int32),
        pltpu.VMEM((KBUF, CH), jnp.int32),
        pltpu.VMEM((KBUF, CH, DH), jnp.float32),
        pltpu.VMEM_SHARED((ACC_ROWS, DH), jnp.float32),
        pltpu.SemaphoreType.DMA,
        pltpu.SemaphoreType.DMA,
        pltpu.SemaphoreType.DMA,
        pltpu.SemaphoreType.DMA,
        pltpu.SemaphoreType.DMA,
        pltpu.SemaphoreType.DMA,
        pltpu.SemaphoreType.DMA,
        pltpu.SemaphoreType.DMA,
        pltpu.SemaphoreType.DMA,
        pltpu.SemaphoreType.DMA,
        pltpu.SemaphoreType.DMA,
        pltpu.SemaphoreType.DMA,
    ],
)
def _msg_kernel(
    hp_hbm, src_hbm, dst_hbm, z_hbm, acc_out,
    srcb, dstb, rows, acc_s,
    l0, l1, l2, g0, g1, g2, s0, s1, s2, d0, d1, d2,
):
    ls = (l0, l1, l2)
    gs = (g0, g1, g2)
    ss = (s0, s1, s2)
    dsems = (d0, d1, d2)
    c = lax.axis_index("c")
    s = lax.axis_index("s")
    pltpu.sync_copy(z_hbm, rows.at[0])
    pltpu.sync_copy(rows.at[0], acc_s.at[pl.ds(s * ACC_STRIPE, CH)])
    pltpu.sync_copy(
        rows.at[0].at[pl.ds(0, STG2)],
        acc_s.at[pl.ds(s * ACC_STRIPE + CH, STG2)],
    )
    plsc.subcore_barrier()
    base = s * EPT
    table = hp_hbm

    def fire_load(k, i):
        off = pl.multiple_of(i * CH, 8)
        pltpu.async_copy(
            src_hbm.at[pl.ds(c * E + base + off, CH)], srcb.at[k], ls[k]
        )
        pltpu.async_copy(dst_hbm.at[pl.ds(base + off, CH)], dstb.at[k], dsems[k])

    def fire_gather(k):
        pltpu.make_async_copy(
            src_hbm.at[pl.ds(0, CH)], srcb.at[k], ls[k]
        ).wait()
        pltpu.async_copy(table.at[srcb.at[k]], rows.at[k], gs[k])

    def fire_scatter(k):
        pltpu.make_async_copy(dst_hbm.at[pl.ds(0, CH)], dstb.at[k], dsems[k]).wait()
        pltpu.make_async_copy(table.at[srcb.at[k]], rows.at[k], gs[k]).wait()
        pltpu.async_copy(rows.at[k], acc_s.at[dstb.at[k]], ss[k], add=True)

    def wait_scatter(k):
        pltpu.make_async_copy(rows.at[k], acc_s.at[dstb.at[k]], ss[k]).wait()

    for k in range(KBUF):
        fire_load(k, k)
    for k in range(KBUF):
        fire_gather(k)

    def round_body(j, carry):
        a = j * KBUF
        for k in range(KBUF):
            fire_scatter(k)
        for k in range(KBUF):
            wait_scatter(k)
            fire_load(k, a + KBUF + k)
            fire_gather(k)
        return carry

    lax.fori_loop(0, NFULL, round_body, 0)
    tb = (NFULL + 1) * KBUF
    for k in range(KBUF):
        fire_scatter(k)
    for k in range(KBUF):
        wait_scatter(k)
        if k < NTAIL:
            fire_load(k, tb + k)
            fire_gather(k)
    for k in range(NTAIL):
        fire_scatter(k)
    for k in range(NTAIL):
        wait_scatter(k)
    plsc.subcore_barrier()
    pltpu.sync_copy(acc_s.at[pl.ds(s * ACC_STRIPE, CH)], rows.at[0])
    pltpu.sync_copy(rows.at[0], acc_out.at[c, pl.ds(s * ACC_STRIPE, CH)])
    pltpu.sync_copy(
        acc_s.at[pl.ds(s * ACC_STRIPE + CH, STG2)],
        rows.at[0].at[pl.ds(0, STG2)],
    )
    pltpu.sync_copy(
        rows.at[0].at[pl.ds(0, STG2)],
        acc_out.at[c, pl.ds(s * ACC_STRIPE + CH, STG2)],
    )


def _repack_body(eil_ref, s0, d0, s1, d1, s2, d2):
    pid = pl.program_id(0)
    outs = ((s0, d0), (s1, d1), (s2, d2))
    for l in range(3):
        @pl.when(pid == l)
        def _(l=l):
            s2x = eil_ref[0, 0, :] * 2
            outs[l][0][0:E] = s2x
            outs[l][0][E : 2 * E] = s2x + 1
            outs[l][1][...] = eil_ref[0, 1, :]


def _repack_call(eil):
    sflat = jax.ShapeDtypeStruct((2 * E,), jnp.int32)
    dflat = jax.ShapeDtypeStruct((E,), jnp.int32)
    return pl.pallas_call(
        _repack_body,
        grid=(3,),
        in_specs=[pl.BlockSpec((1, 2, E), lambda l: (l, 0, 0))],
        out_specs=[
            pl.BlockSpec((2 * E,), lambda l: (0,)),
            pl.BlockSpec((E,), lambda l: (0,)),
        ] * 3,
        out_shape=[sflat, dflat] * 3,
    )(eil)


BM = 2000


def _split(h):
    return h[:, :DH], h[:, DH:]


def _mm1_body(x_ref, w_ref, out_ref):
    out_ref[...] = jnp.dot(
        x_ref[...], w_ref[...], preferred_element_type=jnp.float32
    )


def _mm1_call(x, W1):
    return pl.pallas_call(
        _mm1_body,
        grid=(N // BM,),
        in_specs=[
            pl.BlockSpec((BM, D), lambda j: (j, 0)),
            pl.BlockSpec((D, D), lambda j: (0, 0)),
        ],
        out_specs=pl.BlockSpec((BM, D), lambda j: (j, 0)),
        out_shape=jax.ShapeDtypeStruct((N, D), jnp.float32),
    )(x, W1)


def _disscale_body(deg8_ref, h1_ref, dis_ref, hp_ref):
    d = jnp.sum(deg8_ref[...], axis=(0, 3)) + 1.0
    dis_t = lax.rsqrt(d)
    dis_ref[...] = dis_t.T
    hp_ref[...] = h1_ref[...] * dis_t[0:1, :].T


def _disscale_call(deg8, h1):
    return pl.pallas_call(
        _disscale_body,
        grid=(N // BM,),
        in_specs=[
            pl.BlockSpec((NC, 3, BM, 8), lambda j: (0, 0, j, 0)),
            pl.BlockSpec((BM, D), lambda j: (j, 0)),
        ],
        out_specs=[
            pl.BlockSpec((BM, 3), lambda j: (j, 0)),
            pl.BlockSpec((BM, D), lambda j: (j, 0)),
        ],
        out_shape=[
            jax.ShapeDtypeStruct((N, 3), jnp.float32),
            jax.ShapeDtypeStruct((N, D), jnp.float32),
        ],
    )(deg8, h1)


def _mid_body(l, acc_ref, hp_ref, dis_ref, b_ref, w_ref, out_ref):
    t = jnp.concatenate([acc_ref[0], acc_ref[1]], axis=1) + hp_ref[...]
    t = t * dis_ref[:, l : l + 1] + b_ref[...]
    t = jnp.maximum(t, 0.0)
    h = jnp.dot(t, w_ref[...], preferred_element_type=jnp.float32)
    out_ref[...] = h * dis_ref[:, l + 1 : l + 2]


def _mid_call(l, acc, hp, dis, b, Wn):
    return pl.pallas_call(
        functools.partial(_mid_body, l),
        grid=(N // BM,),
        in_specs=[
            pl.BlockSpec((NC, BM, DH), lambda j: (0, j, 0)),
            pl.BlockSpec((BM, D), lambda j: (j, 0)),
            pl.BlockSpec((BM, 3), lambda j: (j, 0)),
            pl.BlockSpec((1, D), lambda j: (0, 0)),
            pl.BlockSpec((D, D), lambda j: (0, 0)),
        ],
        out_specs=pl.BlockSpec((BM, D), lambda j: (j, 0)),
        out_shape=jax.ShapeDtypeStruct((N, D), jnp.float32),
    )(acc, hp, dis, b, Wn)


def _last_body(acc_ref, hp_ref, dis_ref, b_ref, out_ref):
    t = jnp.concatenate([acc_ref[0], acc_ref[1]], axis=1) + hp_ref[...]
    t = t * dis_ref[:, 2:3] + b_ref[...]
    n2 = jnp.sum(t * t, axis=1, keepdims=True)
    out_ref[...] = t * lax.rsqrt(jnp.maximum(n2, 1e-24))


def _last_call(acc, hp, dis, b):
    return pl.pallas_call(
        _last_body,
        grid=(N // BM,),
        in_specs=[
            pl.BlockSpec((NC, BM, DH), lambda j: (0, j, 0)),
            pl.BlockSpec((BM, D), lambda j: (j, 0)),
            pl.BlockSpec((BM, 3), lambda j: (j, 0)),
            pl.BlockSpec((1, D), lambda j: (0, 0)),
        ],
        out_specs=pl.BlockSpec((BM, D), lambda j: (j, 0)),
        out_shape=jax.ShapeDtypeStruct((N, D), jnp.float32),
    )(acc, hp, dis, b)


def kernel(edge_index_list, x, W1, b1, W2, b2, W3, b3):
    s0, d0, s1, d1, s2, d2 = _repack_call(edge_index_list)
    eighth = jnp.full((CHD, 8), 0.125, jnp.float32)
    z8 = jnp.zeros((DEG_STR, 8), jnp.float32)
    zrows = jnp.zeros((CH, DH), jnp.float32)

    h1 = _mm1_call(x, W1)
    deg8 = _deg_kernel(d0, d1, d2, eighth, z8)
    dis, hp = _disscale_call(deg8, h1)
    acc = _msg_kernel(hp.reshape(2 * N, DH), s0, d0, zrows)
    hp = _mid_call(0, acc, hp, dis, b1.reshape(1, D), W2)
    acc = _msg_kernel(hp.reshape(2 * N, DH), s1, d1, zrows)
    hp = _mid_call(1, acc, hp, dis, b2.reshape(1, D), W3)
    acc = _msg_kernel(hp.reshape(2 * N, DH), s2, d2, zrows)
    return _last_call(acc, hp, dis, b3.reshape(1, D))
